# Initial kernel scaffold; baseline (speedup 1.0000x reference)
import jax
import jax.numpy as jnp
from jax.experimental import pallas as pl


def kernel(X, edge_index, W_l, b_l, W_r):
    raise NotImplementedError("write your pallas kernel here")


# speedup vs baseline: 3.5297x; 3.5297x over previous
import functools

import jax
import jax.numpy as jnp
from jax import lax
from jax.experimental import pallas as pl
from jax.experimental.pallas import tpu as pltpu
from jax.experimental.pallas import tpu_sc as plsc

N = 10000
E = 320000
D = 128

NC = 2
NS = 16
L = 16


def _geom(n, e):
    c_edges = 80
    n_chunks = e // c_edges
    zr = c_edges
    rows_per_tile = (n + NS * zr - 1) // (NS * zr) * zr
    np_ = rows_per_tile * NS
    n_zcopy = rows_per_tile // zr
    return c_edges, n_chunks, zr, rows_per_tile, np_, n_zcopy


def _make_segsum(n, e, d):
    c_edges, n_chunks, zr, rows_per_tile, np_, n_zcopy = _geom(n, e)
    chunks_per_sc = n_chunks // NC
    chunks_per_tile = chunks_per_sc // NS
    assert chunks_per_tile * NS * NC * c_edges == e

    mesh = plsc.VectorSubcoreMesh(core_axis_name="c", subcore_axis_name="s",
                                  num_cores=NC, num_subcores=NS)
    out_type = [jax.ShapeDtypeStruct((NC, np_, d), jnp.float32)]
    scratch = [
        pltpu.VMEM((c_edges,), jnp.int32),
        pltpu.VMEM((c_edges,), jnp.int32),
        pltpu.VMEM((c_edges, d), jnp.float32),
        pltpu.VMEM_SHARED((np_, d), jnp.float32),
        pltpu.SemaphoreType.DMA,
    ]

    def body(t_hbm, g_hbm, s_hbm, out_hbm, gi_v, si_v, rows_v, acc_sh, sem):
        c = lax.axis_index("c")
        s = lax.axis_index("s")
        z16 = jnp.zeros((L,), jnp.float32)

        def zrow(i, _):
            for j in range(d // L):
                rows_v[i, pl.ds(j * L, L)] = z16
            return 0

        lax.fori_loop(0, c_edges, zrow, 0)

        r0 = s * rows_per_tile
        for k in range(n_zcopy):
            pltpu.sync_copy(rows_v, acc_sh.at[pl.ds(r0 + k * zr, zr)])
        plsc.subcore_barrier()

        base_chunk = c * chunks_per_sc + s

        def chunk_body(k, _):
            chunk = base_chunk + k * NS
            off = pl.multiple_of(chunk * c_edges, 8)
            pltpu.sync_copy(g_hbm.at[pl.ds(off, c_edges)], gi_v)
            pltpu.sync_copy(s_hbm.at[pl.ds(off, c_edges)], si_v)
            pltpu.async_copy(t_hbm.at[gi_v], rows_v, sem).wait()
            pltpu.sync_copy(rows_v, acc_sh.at[si_v], add=True)
            return 0

        lax.fori_loop(0, chunks_per_tile, chunk_body, 0)
        plsc.subcore_barrier()

        for k in range(n_zcopy):
            r = r0 + k * zr
            pltpu.sync_copy(acc_sh.at[pl.ds(r, zr)], rows_v)
            pltpu.sync_copy(rows_v, out_hbm.at[c, pl.ds(r, zr)])

    return pl.kernel(body, out_type=out_type, mesh=mesh,
                     scratch_types=scratch)


def _make_counts(n, e, d):
    c_edges, n_chunks, zr, rows_per_tile, np_, n_zcopy = _geom(n, e)
    chunks_per_tile = n_chunks // NS

    mesh = plsc.VectorSubcoreMesh(core_axis_name="c", subcore_axis_name="s",
                                  num_cores=NC, num_subcores=NS)
    out_type = [jax.ShapeDtypeStruct((NC, np_, d), jnp.float32)]
    scratch = [
        pltpu.VMEM((c_edges,), jnp.int32),
        pltpu.VMEM((c_edges, d), jnp.float32),
        pltpu.VMEM_SHARED((np_, d), jnp.float32),
    ]

    def body(dst_hbm, src_hbm, out_hbm, si_v, ones_v, cnt_sh):
        c = lax.axis_index("c")
        s = lax.axis_index("s")
        z16 = jnp.zeros((L,), jnp.float32)

        def zrow(i, _):
            for j in range(d // L):
                ones_v[i, pl.ds(j * L, L)] = z16
            return 0

        lax.fori_loop(0, c_edges, zrow, 0)
        r0 = s * rows_per_tile
        for k in range(n_zcopy):
            pltpu.sync_copy(ones_v, cnt_sh.at[pl.ds(r0 + k * zr, zr)])
        plsc.subcore_barrier()

        one16 = jnp.ones((L,), jnp.float32)

        def orow(i, _):
            for j in range(d // L):
                ones_v[i, pl.ds(j * L, L)] = one16
            return 0

        lax.fori_loop(0, c_edges, orow, 0)

        def count_loop(idx_hbm):
            def cb(k, _):
                chunk = k * NS + s
                off = pl.multiple_of(chunk * c_edges, 8)
                pltpu.sync_copy(idx_hbm.at[pl.ds(off, c_edges)], si_v)
                pltpu.sync_copy(ones_v, cnt_sh.at[si_v], add=True)
                return 0
            lax.fori_loop(0, chunks_per_tile, cb, 0)

        @pl.when(c == 0)
        def _():
            count_loop(dst_hbm)

        @pl.when(c == 1)
        def _():
            count_loop(src_hbm)

        plsc.subcore_barrier()
        for k in range(n_zcopy):
            r = r0 + k * zr
            pltpu.sync_copy(cnt_sh.at[pl.ds(r, zr)], ones_v)
            pltpu.sync_copy(ones_v, out_hbm.at[c, pl.ds(r, zr)])

    return pl.kernel(body, out_type=out_type, mesh=mesh,
                     scratch_types=scratch)


def _make_tc1(n, d, blk):
    grid = (n // blk,)

    def tc1(a_ref, c_ref, x_ref, wl_ref, bl_ref, wr_ref, h_ref, h2_ref):
        asum = a_ref[0] + a_ref[1]
        cnt = c_ref[0, :, 0:1]
        agg = asum / jnp.maximum(cnt, 1.0)
        dn = (((1,), (1,)), ((), ()))
        h = lax.dot_general(agg, wl_ref[...], dn,
                            preferred_element_type=jnp.float32)
        h = h + bl_ref[...]
        h = h + lax.dot_general(x_ref[...], wr_ref[...], dn,
                                preferred_element_type=jnp.float32)
        h = jnp.maximum(h, 0.0)
        h_ref[...] = h
        h2_ref[...] = h * h

    np_ = _geom(n, E)[4]
    return pl.pallas_call(
        tc1,
        grid=grid,
        in_specs=[
            pl.BlockSpec((NC, blk, d), lambda i: (0, i, 0)),
            pl.BlockSpec((1, blk, d), lambda i: (0, i, 0)),
            pl.BlockSpec((blk, d), lambda i: (i, 0)),
            pl.BlockSpec((d, d), lambda i: (0, 0)),
            pl.BlockSpec((1, d), lambda i: (0, 0)),
            pl.BlockSpec((d, d), lambda i: (0, 0)),
        ],
        out_specs=[
            pl.BlockSpec((blk, d), lambda i: (i, 0)),
            pl.BlockSpec((blk, d), lambda i: (i, 0)),
        ],
        out_shape=[
            jax.ShapeDtypeStruct((n, d), jnp.float32),
            jax.ShapeDtypeStruct((n, d), jnp.float32),
        ],
    )


def _make_tc2(n, d, blk):
    grid = (n // blk,)

    def tc2(s1_ref, s2_ref, gc_ref, h_ref, h2_ref, o_ref):
        s1 = s1_ref[0] + s1_ref[1]
        s2 = s2_ref[0] + s2_ref[1]
        g = gc_ref[0, :, 0:1]
        h = h_ref[...]
        gg = (g * h2_ref[...] - 2.0 * h * s1 + s2) / jnp.maximum(g, 1.0)
        o_ref[...] = jnp.tanh(gg)

    return pl.pallas_call(
        tc2,
        grid=grid,
        in_specs=[
            pl.BlockSpec((NC, blk, d), lambda i: (0, i, 0)),
            pl.BlockSpec((NC, blk, d), lambda i: (0, i, 0)),
            pl.BlockSpec((1, blk, d), lambda i: (1, i, 0)),
            pl.BlockSpec((blk, d), lambda i: (i, 0)),
            pl.BlockSpec((blk, d), lambda i: (i, 0)),
        ],
        out_specs=pl.BlockSpec((blk, d), lambda i: (i, 0)),
        out_shape=jax.ShapeDtypeStruct((n, d), jnp.float32),
    )


_make_segsum_c = functools.cache(_make_segsum)
_make_counts_c = functools.cache(_make_counts)
_make_tc1_c = functools.cache(_make_tc1)
_make_tc2_c = functools.cache(_make_tc2)


def kernel(X, edge_index, W_l, b_l, W_r):
    src = edge_index[0]
    dst = edge_index[1]
    CNT, = _make_counts_c(N, E, D)(dst, src)
    P, = _make_segsum_c(N, E, D)(X, src, dst)
    H, H2 = _make_tc1_c(N, D, 1000)(P, CNT, X, W_l, b_l.reshape(1, D), W_r)
    S1, = _make_segsum_c(N, E, D)(H, dst, src)
    S2, = _make_segsum_c(N, E, D)(H2, dst, src)
    return _make_tc2_c(N, D, 1000)(S1, S2, CNT, H, H2)


# speedup vs baseline: 9.3075x; 2.6369x over previous
import functools

import jax
import jax.numpy as jnp
from jax import lax
from jax.experimental import pallas as pl
from jax.experimental.pallas import tpu as pltpu
from jax.experimental.pallas import tpu_sc as plsc

N = 10000
E = 320000
D = 128

NC = 2
NS = 16
L = 16
C_E = 80
NP = 10240
RPT = NP // NS
ZR = C_E
NZC = RPT // ZR
EPT = E // (NC * NS)
SB = 25
SE = SB * C_E
NSTG = EPT // SE


def _mesh():
    return plsc.VectorSubcoreMesh(core_axis_name="c", subcore_axis_name="s",
                                  num_cores=NC, num_subcores=NS)


def _make_segsum(n, e, d):
    assert NSTG * SE * NC * NS == e

    out_type = [jax.ShapeDtypeStruct((NC, NP, d), jnp.float32)]
    scratch = [
        pltpu.VMEM((SE,), jnp.int32),
        pltpu.VMEM((2, C_E), jnp.int32),
        pltpu.VMEM((2, C_E, d), jnp.float32),
        pltpu.VMEM_SHARED((NP, d), jnp.float32),
        pltpu.SemaphoreType.DMA,
        pltpu.SemaphoreType.DMA,
        pltpu.SemaphoreType.DMA,
        pltpu.SemaphoreType.DMA,
        pltpu.SemaphoreType.DMA,
        pltpu.SemaphoreType.DMA,
    ]

    def body(t_hbm, g_hbm, s_hbm, out_hbm, gi_v, si2, rows, acc_sh,
             gs0, gs1, is0, is1, ss0, ss1):
        c = lax.axis_index("c")
        s = lax.axis_index("s")
        gsem = (gs0, gs1)
        isem = (is0, is1)
        ssem = (ss0, ss1)
        z16 = jnp.zeros((L,), jnp.float32)

        def zrow(i, _):
            for j in range(d // L):
                rows[0, i, pl.ds(j * L, L)] = z16
            return 0

        lax.fori_loop(0, C_E, zrow, 0)
        r0 = s * RPT
        for k in range(NZC):
            pltpu.sync_copy(rows.at[0], acc_sh.at[pl.ds(r0 + k * ZR, ZR)])
        plsc.subcore_barrier()

        wid = c * NS + s
        base_e = wid * EPT
        scat_desc = [None, None]
        for stg in range(NSTG):
            eoff = pl.multiple_of(base_e + stg * SE, 8)
            pltpu.sync_copy(g_hbm.at[pl.ds(eoff, SE)], gi_v)
            si_d0 = pltpu.async_copy(
                s_hbm.at[pl.ds(eoff, C_E)], si2.at[0], isem[0])
            g_d0 = pltpu.async_copy(
                t_hbm.at[gi_v.at[pl.ds(0, C_E)]], rows.at[0], gsem[0])
            g_desc = {0: g_d0}
            si_desc = {0: si_d0}
            for k in range(SB):
                b = k % 2
                nb = (k + 1) % 2
                if k + 1 < SB:
                    if scat_desc[nb] is not None:
                        scat_desc[nb].wait()
                        scat_desc[nb] = None
                    si_desc[k + 1] = pltpu.async_copy(
                        s_hbm.at[pl.ds(eoff + (k + 1) * C_E, C_E)],
                        si2.at[nb], isem[nb])
                    g_desc[k + 1] = pltpu.async_copy(
                        t_hbm.at[gi_v.at[pl.ds((k + 1) * C_E, C_E)]],
                        rows.at[nb], gsem[nb])
                g_desc[k].wait()
                si_desc[k].wait()
                if scat_desc[b] is not None:
                    scat_desc[b].wait()
                scat_desc[b] = pltpu.async_copy(
                    rows.at[b], acc_sh.at[si2.at[b]], ssem[b], add=True)
            for b in range(2):
                if scat_desc[b] is not None:
                    scat_desc[b].wait()
                    scat_desc[b] = None

        plsc.subcore_barrier()
        for k in range(NZC):
            r = r0 + k * ZR
            pltpu.sync_copy(acc_sh.at[pl.ds(r, ZR)], rows.at[0])
            pltpu.sync_copy(rows.at[0], out_hbm.at[c, pl.ds(r, ZR)])

    return pl.kernel(body, out_type=out_type, mesh=_mesh(),
                     scratch_types=scratch)


def _make_counts(n, e):
    out_type = [jax.ShapeDtypeStruct((2, NC * NS, NP), jnp.float32)]
    scratch = [
        pltpu.VMEM((SE,), jnp.int32),
        pltpu.VMEM((SE,), jnp.int32),
        pltpu.VMEM((NP,), jnp.float32),
        pltpu.VMEM((NP,), jnp.float32),
    ]

    def body(g_hbm, s_hbm, out_hbm, gi_v, si_v, csrc, cdst):
        c = lax.axis_index("c")
        s = lax.axis_index("s")
        z16 = jnp.zeros((L,), jnp.float32)
        one16 = jnp.ones((L,), jnp.float32)

        def zrow(i, _):
            csrc[pl.ds(i * L, L)] = z16
            cdst[pl.ds(i * L, L)] = z16
            return 0

        lax.fori_loop(0, NP // L, zrow, 0)

        wid = c * NS + s
        base = wid * EPT
        for stg in range(NSTG):
            off = pl.multiple_of(base + stg * SE, 8)
            pltpu.sync_copy(g_hbm.at[pl.ds(off, SE)], gi_v)
            pltpu.sync_copy(s_hbm.at[pl.ds(off, SE)], si_v)

            def vec_body(j, _):
                gi = gi_v[pl.ds(j * L, L)]
                si = si_v[pl.ds(j * L, L)]
                plsc.addupdate_scatter(csrc, [gi], one16)
                plsc.addupdate_scatter(cdst, [si], one16)
                return 0

            lax.fori_loop(0, SE // L, vec_body, 0)
        pltpu.sync_copy(csrc, out_hbm.at[0, wid])
        pltpu.sync_copy(cdst, out_hbm.at[1, wid])

    return pl.kernel(body, out_type=out_type, mesh=_mesh(),
                     scratch_types=scratch,
                     compiler_params=pltpu.CompilerParams(
                         needs_layout_passes=False))


def _col_reduce(c_ref, blk):
    m = c_ref[...].reshape(NC * NS, blk)
    ones = jnp.ones((NC * NS, 1), jnp.float32)
    return lax.dot_general(m, ones, (((0,), (0,)), ((), ())),
                           preferred_element_type=jnp.float32)


def _make_tc1(n, d, blk):
    grid = ((n + blk - 1) // blk,)

    def tc1(a_ref, c_ref, x_ref, wl_ref, bl_ref, wr_ref, h_ref, h2_ref):
        asum = a_ref[0] + a_ref[1]
        cnt = _col_reduce(c_ref, blk)
        agg = asum / jnp.maximum(cnt, 1.0)
        dn = (((1,), (1,)), ((), ()))
        h = lax.dot_general(agg, wl_ref[...], dn,
                            preferred_element_type=jnp.float32)
        h = h + bl_ref[...]
        h = h + lax.dot_general(x_ref[...], wr_ref[...], dn,
                                preferred_element_type=jnp.float32)
        h = jnp.maximum(h, 0.0)
        h_ref[...] = h
        h2_ref[...] = h * h

    return pl.pallas_call(
        tc1,
        grid=grid,
        in_specs=[
            pl.BlockSpec((NC, blk, d), lambda i: (0, i, 0)),
            pl.BlockSpec((1, NC * NS, blk), lambda i: (1, 0, i)),
            pl.BlockSpec((blk, d), lambda i: (i, 0)),
            pl.BlockSpec((d, d), lambda i: (0, 0)),
            pl.BlockSpec((1, d), lambda i: (0, 0)),
            pl.BlockSpec((d, d), lambda i: (0, 0)),
        ],
        out_specs=[
            pl.BlockSpec((blk, d), lambda i: (i, 0)),
            pl.BlockSpec((blk, d), lambda i: (i, 0)),
        ],
        out_shape=[
            jax.ShapeDtypeStruct((n, d), jnp.float32),
            jax.ShapeDtypeStruct((n, d), jnp.float32),
        ],
    )


def _make_tc2(n, d, blk):
    grid = ((n + blk - 1) // blk,)

    def tc2(s1_ref, s2_ref, gc_ref, h_ref, h2_ref, o_ref):
        s1 = s1_ref[0] + s1_ref[1]
        s2 = s2_ref[0] + s2_ref[1]
        g = _col_reduce(gc_ref, blk)
        h = h_ref[...]
        gg = (g * h2_ref[...] - 2.0 * h * s1 + s2) / jnp.maximum(g, 1.0)
        o_ref[...] = jnp.tanh(gg)

    return pl.pallas_call(
        tc2,
        grid=grid,
        in_specs=[
            pl.BlockSpec((NC, blk, d), lambda i: (0, i, 0)),
            pl.BlockSpec((NC, blk, d), lambda i: (0, i, 0)),
            pl.BlockSpec((1, NC * NS, blk), lambda i: (0, 0, i)),
            pl.BlockSpec((blk, d), lambda i: (i, 0)),
            pl.BlockSpec((blk, d), lambda i: (i, 0)),
        ],
        out_specs=pl.BlockSpec((blk, d), lambda i: (i, 0)),
        out_shape=jax.ShapeDtypeStruct((n, d), jnp.float32),
    )


_make_segsum_c = functools.cache(_make_segsum)
_make_counts_c = functools.cache(_make_counts)
_make_tc1_c = functools.cache(_make_tc1)
_make_tc2_c = functools.cache(_make_tc2)


def kernel(X, edge_index, W_l, b_l, W_r):
    src = edge_index[0]
    dst = edge_index[1]
    CNT, = _make_counts_c(N, E)(src, dst)
    P, = _make_segsum_c(N, E, D)(X, src, dst)
    H, H2 = _make_tc1_c(N, D, 1024)(P, CNT, X, W_l, b_l.reshape(1, D), W_r)
    S1, = _make_segsum_c(N, E, D)(H, dst, src)
    S2, = _make_segsum_c(N, E, D)(H2, dst, src)
    return _make_tc2_c(N, D, 1024)(S1, S2, CNT, H, H2)


# speedup vs baseline: 10.9867x; 1.1804x over previous
import functools

import jax
import jax.numpy as jnp
from jax import lax
from jax.experimental import pallas as pl
from jax.experimental.pallas import tpu as pltpu
from jax.experimental.pallas import tpu_sc as plsc

N = 10000
E = 320000
D = 128

NC = 2
NS = 16
L = 16
C_E = 80
NB = 3
NP = 10112
RPT = NP // NS
NPC = 10240
EPT = E // (NC * NS)
SB = 25
SE = SB * C_E
NSTG = EPT // SE
_WO = [(k * C_E, C_E) for k in range(RPT // C_E)] + [
    (RPT - RPT % C_E, RPT % C_E)] if RPT % C_E else \
    [(k * C_E, C_E) for k in range(RPT // C_E)]


def _mesh():
    return plsc.VectorSubcoreMesh(core_axis_name="c", subcore_axis_name="s",
                                  num_cores=NC, num_subcores=NS)


def _segsum_phase(t_hbm, g_hbm, s_hbm, out_hbm, gi_v, si2, rows, acc_sh,
                  gsem, isem, ssem, c, s):
    z16 = jnp.zeros((L,), jnp.float32)

    def zrow(i, _):
        for j in range(D // L):
            rows[0, i, pl.ds(j * L, L)] = z16
        return 0

    lax.fori_loop(0, C_E, zrow, 0)
    r0 = s * RPT
    for off, sz in _WO:
        pltpu.sync_copy(rows.at[0, pl.ds(0, sz)],
                        acc_sh.at[pl.ds(r0 + off, sz)])
    plsc.subcore_barrier()

    wid = c * NS + s
    base_e = wid * EPT
    scat_desc = [None] * NB
    for stg in range(NSTG):
        eoff = pl.multiple_of(base_e + stg * SE, 8)
        pltpu.sync_copy(g_hbm.at[pl.ds(eoff, SE)], gi_v)
        g_desc = {}
        si_desc = {}

        def fire(k):
            b = k % NB
            si_desc[k] = pltpu.async_copy(
                s_hbm.at[pl.ds(eoff + k * C_E, C_E)], si2.at[b], isem[b])
            g_desc[k] = pltpu.async_copy(
                t_hbm.at[gi_v.at[pl.ds(k * C_E, C_E)]], rows.at[b], gsem[b])

        fire(0)
        if SB > 1:
            fire(1)
        for k in range(SB):
            b = k % NB
            if k + 2 < SB:
                nb = (k + 2) % NB
                if scat_desc[nb] is not None:
                    scat_desc[nb].wait()
                    scat_desc[nb] = None
                fire(k + 2)
            g_desc[k].wait()
            si_desc[k].wait()
            if scat_desc[b] is not None:
                scat_desc[b].wait()
            scat_desc[b] = pltpu.async_copy(
                rows.at[b], acc_sh.at[si2.at[b]], ssem[b], add=True)
        for b in range(NB):
            if scat_desc[b] is not None:
                scat_desc[b].wait()
                scat_desc[b] = None

    plsc.subcore_barrier()
    for off, sz in _WO:
        r = r0 + off
        pltpu.sync_copy(acc_sh.at[pl.ds(r, sz)], rows.at[0, pl.ds(0, sz)])
        pltpu.sync_copy(rows.at[0, pl.ds(0, sz)],
                        out_hbm.at[c, pl.ds(r, sz)])


def _segsum_scratch():
    return [
        pltpu.VMEM((SE,), jnp.int32),
        pltpu.VMEM((NB, C_E), jnp.int32),
        pltpu.VMEM((NB, C_E, D), jnp.float32),
        pltpu.VMEM_SHARED((NP, D), jnp.float32),
    ] + [pltpu.SemaphoreType.DMA] * (3 * NB)


def _make_segsum(n, e, d):
    assert NSTG * SE * NC * NS == e
    out_type = [jax.ShapeDtypeStruct((NC, NP, d), jnp.float32)]

    def body(t_hbm, g_hbm, s_hbm, out_hbm, gi_v, si2, rows, acc_sh, *sems):
        c = lax.axis_index("c")
        s = lax.axis_index("s")
        _segsum_phase(t_hbm, g_hbm, s_hbm, out_hbm, gi_v, si2, rows, acc_sh,
                      sems[0:NB], sems[NB:2 * NB], sems[2 * NB:3 * NB], c, s)

    return pl.kernel(body, out_type=out_type, mesh=_mesh(),
                     scratch_types=_segsum_scratch())


def _make_segsum2(n, e, d):
    out_type = [jax.ShapeDtypeStruct((NC, NP, d), jnp.float32),
                jax.ShapeDtypeStruct((NC, NP, d), jnp.float32)]

    def body(t1_hbm, t2_hbm, g_hbm, s_hbm, o1_hbm, o2_hbm,
             gi_v, si2, rows, acc_sh, *sems):
        c = lax.axis_index("c")
        s = lax.axis_index("s")
        for t_hbm, out_hbm in ((t1_hbm, o1_hbm), (t2_hbm, o2_hbm)):
            _segsum_phase(t_hbm, g_hbm, s_hbm, out_hbm, gi_v, si2, rows,
                          acc_sh, sems[0:NB], sems[NB:2 * NB],
                          sems[2 * NB:3 * NB], c, s)
            plsc.subcore_barrier()

    return pl.kernel(body, out_type=out_type, mesh=_mesh(),
                     scratch_types=_segsum_scratch())


def _make_counts(n, e):
    out_type = [jax.ShapeDtypeStruct((2, NC * NS, NPC), jnp.float32)]
    scratch = [
        pltpu.VMEM((SE,), jnp.int32),
        pltpu.VMEM((SE,), jnp.int32),
        pltpu.VMEM((NPC,), jnp.float32),
        pltpu.VMEM((NPC,), jnp.float32),
    ]

    def body(g_hbm, s_hbm, out_hbm, gi_v, si_v, csrc, cdst):
        c = lax.axis_index("c")
        s = lax.axis_index("s")
        z16 = jnp.zeros((L,), jnp.float32)
        one16 = jnp.ones((L,), jnp.float32)

        def zrow(i, _):
            csrc[pl.ds(i * L, L)] = z16
            cdst[pl.ds(i * L, L)] = z16
            return 0

        lax.fori_loop(0, NPC // L, zrow, 0)

        wid = c * NS + s
        base = wid * EPT
        for stg in range(NSTG):
            off = pl.multiple_of(base + stg * SE, 8)
            pltpu.sync_copy(g_hbm.at[pl.ds(off, SE)], gi_v)
            pltpu.sync_copy(s_hbm.at[pl.ds(off, SE)], si_v)

            def vec_body(j, _):
                gi = gi_v[pl.ds(j * L, L)]
                si = si_v[pl.ds(j * L, L)]
                plsc.addupdate_scatter(csrc, [gi], one16)
                plsc.addupdate_scatter(cdst, [si], one16)
                return 0

            lax.fori_loop(0, SE // L, vec_body, 0)
        pltpu.sync_copy(csrc, out_hbm.at[0, wid])
        pltpu.sync_copy(cdst, out_hbm.at[1, wid])

    return pl.kernel(body, out_type=out_type, mesh=_mesh(),
                     scratch_types=scratch,
                     compiler_params=pltpu.CompilerParams(
                         needs_layout_passes=False))


def _col_reduce(c_ref, blk):
    m = c_ref[...].reshape(NC * NS, blk)
    ones = jnp.ones((NC * NS, 1), jnp.float32)
    return lax.dot_general(m, ones, (((0,), (0,)), ((), ())),
                           preferred_element_type=jnp.float32)


def _make_tc1(n, d, blk):
    grid = ((n + blk - 1) // blk,)

    def tc1(a_ref, c_ref, x_ref, wl_ref, bl_ref, wr_ref, h_ref, h2_ref):
        asum = a_ref[0] + a_ref[1]
        cnt = _col_reduce(c_ref, blk)
        agg = asum / jnp.maximum(cnt, 1.0)
        dn = (((1,), (1,)), ((), ()))
        h = lax.dot_general(agg, wl_ref[...], dn,
                            preferred_element_type=jnp.float32)
        h = h + bl_ref[...]
        h = h + lax.dot_general(x_ref[...], wr_ref[...], dn,
                                preferred_element_type=jnp.float32)
        h = jnp.maximum(h, 0.0)
        h_ref[...] = h
        h2_ref[...] = h * h

    return pl.pallas_call(
        tc1,
        grid=grid,
        in_specs=[
            pl.BlockSpec((NC, blk, d), lambda i: (0, i, 0)),
            pl.BlockSpec((1, NC * NS, blk), lambda i: (1, 0, i)),
            pl.BlockSpec((blk, d), lambda i: (i, 0)),
            pl.BlockSpec((d, d), lambda i: (0, 0)),
            pl.BlockSpec((1, d), lambda i: (0, 0)),
            pl.BlockSpec((d, d), lambda i: (0, 0)),
        ],
        out_specs=[
            pl.BlockSpec((blk, d), lambda i: (i, 0)),
            pl.BlockSpec((blk, d), lambda i: (i, 0)),
        ],
        out_shape=[
            jax.ShapeDtypeStruct((n, d), jnp.float32),
            jax.ShapeDtypeStruct((n, d), jnp.float32),
        ],
    )


def _make_tc2(n, d, blk):
    grid = ((n + blk - 1) // blk,)

    def tc2(s1_ref, s2_ref, gc_ref, h_ref, h2_ref, o_ref):
        s1 = s1_ref[0] + s1_ref[1]
        s2 = s2_ref[0] + s2_ref[1]
        g = _col_reduce(gc_ref, blk)
        h = h_ref[...]
        gg = (g * h2_ref[...] - 2.0 * h * s1 + s2) / jnp.maximum(g, 1.0)
        o_ref[...] = jnp.tanh(gg)

    return pl.pallas_call(
        tc2,
        grid=grid,
        in_specs=[
            pl.BlockSpec((NC, blk, d), lambda i: (0, i, 0)),
            pl.BlockSpec((NC, blk, d), lambda i: (0, i, 0)),
            pl.BlockSpec((1, NC * NS, blk), lambda i: (0, 0, i)),
            pl.BlockSpec((blk, d), lambda i: (i, 0)),
            pl.BlockSpec((blk, d), lambda i: (i, 0)),
        ],
        out_specs=pl.BlockSpec((blk, d), lambda i: (i, 0)),
        out_shape=jax.ShapeDtypeStruct((n, d), jnp.float32),
    )


_make_segsum_c = functools.cache(_make_segsum)
_make_segsum2_c = functools.cache(_make_segsum2)
_make_counts_c = functools.cache(_make_counts)
_make_tc1_c = functools.cache(_make_tc1)
_make_tc2_c = functools.cache(_make_tc2)


def kernel(X, edge_index, W_l, b_l, W_r):
    src = edge_index[0]
    dst = edge_index[1]
    CNT, = _make_counts_c(N, E)(src, dst)
    P, = _make_segsum_c(N, E, D)(X, src, dst)
    H, H2 = _make_tc1_c(N, D, 1024)(P, CNT, X, W_l, b_l.reshape(1, D), W_r)
    S1, S2 = _make_segsum2_c(N, E, D)(H, H2, dst, src)
    return _make_tc2_c(N, D, 1024)(S1, S2, CNT, H, H2)


# speedup vs baseline: 11.3915x; 1.0368x over previous
import functools

import jax
import jax.numpy as jnp
from jax import lax
from jax.experimental import pallas as pl
from jax.experimental.pallas import tpu as pltpu
from jax.experimental.pallas import tpu_sc as plsc

N = 10000
E = 320000
D = 128

NC = 2
NS = 16
L = 16
C_E = 80
NB = 3
NP = 10112
RPT = NP // NS
NPC = 10240
EPT = E // (NC * NS)
SB = 25
SE = SB * C_E
NSTG = EPT // SE
_WO = [(k * C_E, C_E) for k in range(RPT // C_E)] + [
    (RPT - RPT % C_E, RPT % C_E)] if RPT % C_E else \
    [(k * C_E, C_E) for k in range(RPT // C_E)]


def _mesh():
    return plsc.VectorSubcoreMesh(core_axis_name="c", subcore_axis_name="s",
                                  num_cores=NC, num_subcores=NS)


def _segsum_phase(t_hbm, g_hbm, s_hbm, out_at, gi_v, si2, rows, acc_sh,
                  gsem, isem, ssem, base_e, nstg, s):
    z16 = jnp.zeros((L,), jnp.float32)

    def zrow(i, _):
        for j in range(D // L):
            rows[0, i, pl.ds(j * L, L)] = z16
        return 0

    lax.fori_loop(0, C_E, zrow, 0)
    r0 = s * RPT
    for off, sz in _WO:
        pltpu.sync_copy(rows.at[0, pl.ds(0, sz)],
                        acc_sh.at[pl.ds(r0 + off, sz)])
    plsc.subcore_barrier()

    scat_desc = [None] * NB
    for stg in range(nstg):
        eoff = pl.multiple_of(base_e + stg * SE, 8)
        pltpu.sync_copy(g_hbm.at[pl.ds(eoff, SE)], gi_v)
        g_desc = {}
        si_desc = {}

        def fire(k):
            b = k % NB
            si_desc[k] = pltpu.async_copy(
                s_hbm.at[pl.ds(eoff + k * C_E, C_E)], si2.at[b], isem[b])
            g_desc[k] = pltpu.async_copy(
                t_hbm.at[gi_v.at[pl.ds(k * C_E, C_E)]], rows.at[b], gsem[b])

        fire(0)
        if SB > 1:
            fire(1)
        for k in range(SB):
            b = k % NB
            if k + 2 < SB:
                nb = (k + 2) % NB
                if scat_desc[nb] is not None:
                    scat_desc[nb].wait()
                    scat_desc[nb] = None
                fire(k + 2)
            g_desc[k].wait()
            si_desc[k].wait()
            if scat_desc[b] is not None:
                scat_desc[b].wait()
            scat_desc[b] = pltpu.async_copy(
                rows.at[b], acc_sh.at[si2.at[b]], ssem[b], add=True)
        for b in range(NB):
            if scat_desc[b] is not None:
                scat_desc[b].wait()
                scat_desc[b] = None

    plsc.subcore_barrier()
    for off, sz in _WO:
        r = r0 + off
        pltpu.sync_copy(acc_sh.at[pl.ds(r, sz)], rows.at[0, pl.ds(0, sz)])
        pltpu.sync_copy(rows.at[0, pl.ds(0, sz)], out_at(r, sz))


def _segsum_scratch():
    return [
        pltpu.VMEM((SE,), jnp.int32),
        pltpu.VMEM((NB, C_E), jnp.int32),
        pltpu.VMEM((NB, C_E, D), jnp.float32),
        pltpu.VMEM_SHARED((NP, D), jnp.float32),
    ] + [pltpu.SemaphoreType.DMA] * (3 * NB)


def _make_segsum(n, e, d):
    assert NSTG * SE * NC * NS == e
    out_type = [jax.ShapeDtypeStruct((NC, NP, d), jnp.float32)]

    def body(t_hbm, g_hbm, s_hbm, out_hbm, gi_v, si2, rows, acc_sh, *sems):
        c = lax.axis_index("c")
        s = lax.axis_index("s")
        base_e = (c * NS + s) * EPT
        _segsum_phase(t_hbm, g_hbm, s_hbm,
                      lambda r, sz: out_hbm.at[c, pl.ds(r, sz)],
                      gi_v, si2, rows, acc_sh,
                      sems[0:NB], sems[NB:2 * NB], sems[2 * NB:3 * NB],
                      base_e, NSTG, s)

    return pl.kernel(body, out_type=out_type, mesh=_mesh(),
                     scratch_types=_segsum_scratch())


def _make_segsum2(n, e, d):
    ept2 = e // NS
    nstg2 = ept2 // SE
    assert nstg2 * SE == ept2
    out_type = [jax.ShapeDtypeStruct((NP, d), jnp.float32),
                jax.ShapeDtypeStruct((NP, d), jnp.float32)]

    def body(t1_hbm, t2_hbm, g_hbm, s_hbm, o1_hbm, o2_hbm,
             gi_v, si2, rows, acc_sh, *sems):
        c = lax.axis_index("c")
        s = lax.axis_index("s")
        base_e = s * ept2

        @pl.when(c == 0)
        def _():
            _segsum_phase(t1_hbm, g_hbm, s_hbm,
                          lambda r, sz: o1_hbm.at[pl.ds(r, sz)],
                          gi_v, si2, rows, acc_sh,
                          sems[0:NB], sems[NB:2 * NB], sems[2 * NB:3 * NB],
                          base_e, nstg2, s)

        @pl.when(c == 1)
        def _():
            _segsum_phase(t2_hbm, g_hbm, s_hbm,
                          lambda r, sz: o2_hbm.at[pl.ds(r, sz)],
                          gi_v, si2, rows, acc_sh,
                          sems[0:NB], sems[NB:2 * NB], sems[2 * NB:3 * NB],
                          base_e, nstg2, s)

    return pl.kernel(body, out_type=out_type, mesh=_mesh(),
                     scratch_types=_segsum_scratch())


def _make_counts(n, e):
    out_type = [jax.ShapeDtypeStruct((2, NC * NS, NPC), jnp.float32)]
    scratch = [
        pltpu.VMEM((SE,), jnp.int32),
        pltpu.VMEM((SE,), jnp.int32),
        pltpu.VMEM((NPC,), jnp.float32),
        pltpu.VMEM((NPC,), jnp.float32),
    ]

    def body(g_hbm, s_hbm, out_hbm, gi_v, si_v, csrc, cdst):
        c = lax.axis_index("c")
        s = lax.axis_index("s")
        z16 = jnp.zeros((L,), jnp.float32)
        one16 = jnp.ones((L,), jnp.float32)

        def zrow(i, _):
            csrc[pl.ds(i * L, L)] = z16
            cdst[pl.ds(i * L, L)] = z16
            return 0

        lax.fori_loop(0, NPC // L, zrow, 0)

        wid = c * NS + s
        base = wid * EPT
        for stg in range(NSTG):
            off = pl.multiple_of(base + stg * SE, 8)
            pltpu.sync_copy(g_hbm.at[pl.ds(off, SE)], gi_v)
            pltpu.sync_copy(s_hbm.at[pl.ds(off, SE)], si_v)

            def vec_body(j, _):
                gi = gi_v[pl.ds(j * L, L)]
                si = si_v[pl.ds(j * L, L)]
                plsc.addupdate_scatter(csrc, [gi], one16)
                plsc.addupdate_scatter(cdst, [si], one16)
                return 0

            lax.fori_loop(0, SE // L, vec_body, 0)
        pltpu.sync_copy(csrc, out_hbm.at[0, wid])
        pltpu.sync_copy(cdst, out_hbm.at[1, wid])

    return pl.kernel(body, out_type=out_type, mesh=_mesh(),
                     scratch_types=scratch,
                     compiler_params=pltpu.CompilerParams(
                         needs_layout_passes=False))


def _col_reduce(c_ref, blk):
    m = c_ref[...].reshape(NC * NS, blk)
    ones = jnp.ones((NC * NS, 1), jnp.float32)
    return lax.dot_general(m, ones, (((0,), (0,)), ((), ())),
                           preferred_element_type=jnp.float32)


def _make_tc1(n, d, blk):
    grid = ((n + blk - 1) // blk,)

    def tc1(a_ref, c_ref, x_ref, wl_ref, bl_ref, wr_ref, h_ref, h2_ref):
        asum = a_ref[0] + a_ref[1]
        cnt = _col_reduce(c_ref, blk)
        agg = asum / jnp.maximum(cnt, 1.0)
        dn = (((1,), (1,)), ((), ()))
        h = lax.dot_general(agg, wl_ref[...], dn,
                            preferred_element_type=jnp.float32)
        h = h + bl_ref[...]
        h = h + lax.dot_general(x_ref[...], wr_ref[...], dn,
                                preferred_element_type=jnp.float32)
        h = jnp.maximum(h, 0.0)
        h_ref[...] = h
        h2_ref[...] = h * h

    return pl.pallas_call(
        tc1,
        grid=grid,
        in_specs=[
            pl.BlockSpec((NC, blk, d), lambda i: (0, i, 0)),
            pl.BlockSpec((1, NC * NS, blk), lambda i: (1, 0, i)),
            pl.BlockSpec((blk, d), lambda i: (i, 0)),
            pl.BlockSpec((d, d), lambda i: (0, 0)),
            pl.BlockSpec((1, d), lambda i: (0, 0)),
            pl.BlockSpec((d, d), lambda i: (0, 0)),
        ],
        out_specs=[
            pl.BlockSpec((blk, d), lambda i: (i, 0)),
            pl.BlockSpec((blk, d), lambda i: (i, 0)),
        ],
        out_shape=[
            jax.ShapeDtypeStruct((n, d), jnp.float32),
            jax.ShapeDtypeStruct((n, d), jnp.float32),
        ],
    )


def _make_tc2(n, d, blk):
    grid = ((n + blk - 1) // blk,)

    def tc2(s1_ref, s2_ref, gc_ref, h_ref, h2_ref, o_ref):
        s1 = s1_ref[...]
        s2 = s2_ref[...]
        g = _col_reduce(gc_ref, blk)
        h = h_ref[...]
        gg = (g * h2_ref[...] - 2.0 * h * s1 + s2) / jnp.maximum(g, 1.0)
        o_ref[...] = jnp.tanh(gg)

    return pl.pallas_call(
        tc2,
        grid=grid,
        in_specs=[
            pl.BlockSpec((blk, d), lambda i: (i, 0)),
            pl.BlockSpec((blk, d), lambda i: (i, 0)),
            pl.BlockSpec((1, NC * NS, blk), lambda i: (0, 0, i)),
            pl.BlockSpec((blk, d), lambda i: (i, 0)),
            pl.BlockSpec((blk, d), lambda i: (i, 0)),
        ],
        out_specs=pl.BlockSpec((blk, d), lambda i: (i, 0)),
        out_shape=jax.ShapeDtypeStruct((n, d), jnp.float32),
    )


_make_segsum_c = functools.cache(_make_segsum)
_make_segsum2_c = functools.cache(_make_segsum2)
_make_counts_c = functools.cache(_make_counts)
_make_tc1_c = functools.cache(_make_tc1)
_make_tc2_c = functools.cache(_make_tc2)


def kernel(X, edge_index, W_l, b_l, W_r):
    src = edge_index[0]
    dst = edge_index[1]
    CNT, = _make_counts_c(N, E)(src, dst)
    P, = _make_segsum_c(N, E, D)(X, src, dst)
    H, H2 = _make_tc1_c(N, D, 1024)(P, CNT, X, W_l, b_l.reshape(1, D), W_r)
    S1, S2 = _make_segsum2_c(N, E, D)(H, H2, dst, src)
    return _make_tc2_c(N, D, 1024)(S1, S2, CNT, H, H2)


# speedup vs baseline: 12.1640x; 1.0678x over previous
import functools

import jax
import jax.numpy as jnp
from jax import lax
from jax.experimental import pallas as pl
from jax.experimental.pallas import tpu as pltpu
from jax.experimental.pallas import tpu_sc as plsc

N = 10000
E = 320000
D = 128

NC = 2
NS = 16
L = 16
C_E = 80
NB = 3
NI = 4
NP = 10112
RPT = NP // NS
NPC = 10240
EPT = E // (NC * NS)
SB = 25
SE = SB * C_E
NSTG = EPT // SE
_WO = [(k * C_E, C_E) for k in range(RPT // C_E)] + [
    (RPT - RPT % C_E, RPT % C_E)] if RPT % C_E else \
    [(k * C_E, C_E) for k in range(RPT // C_E)]


def _mesh():
    return plsc.VectorSubcoreMesh(core_axis_name="c", subcore_axis_name="s",
                                  num_cores=NC, num_subcores=NS)


def _segsum_phase(t_hbm, g_hbm, s_hbm, out_at, gi4, si4, rows, acc_sh,
                  gsem, isem, ssem, base_e, nch, s):
    z16 = jnp.zeros((L,), jnp.float32)

    def zrow(i, _):
        for j in range(D // L):
            rows[0, i, pl.ds(j * L, L)] = z16
        return 0

    lax.fori_loop(0, C_E, zrow, 0)
    r0 = s * RPT
    for off, sz in _WO:
        pltpu.sync_copy(rows.at[0, pl.ds(0, sz)],
                        acc_sh.at[pl.ds(r0 + off, sz)])
    plsc.subcore_barrier()

    gi_desc, si_desc, g_desc, scat_desc = {}, {}, {}, {}

    def fire_idx(j):
        b = j % NI
        eoff = pl.multiple_of(base_e + j * C_E, 8)
        gi_desc[j] = pltpu.async_copy(
            g_hbm.at[pl.ds(eoff, C_E)], gi4.at[b], gsem[b])
        si_desc[j] = pltpu.async_copy(
            s_hbm.at[pl.ds(eoff, C_E)], si4.at[b], gsem[NI + b])

    def fire_gather(j):
        gi_desc.pop(j).wait()
        g_desc[j] = pltpu.async_copy(
            t_hbm.at[gi4.at[j % NI]], rows.at[j % NB], isem[j % NB])

    def wait_scat(j):
        if j in scat_desc:
            scat_desc.pop(j).wait()

    for j in range(min(NI - 1, nch)):
        fire_idx(j)
    for j in range(min(2, nch)):
        fire_gather(j)
    for m in range(nch):
        wait_scat(m - 1)
        if m + NI - 1 < nch:
            fire_idx(m + NI - 1)
        if m + 2 < nch:
            fire_gather(m + 2)
        g_desc.pop(m).wait()
        si_desc.pop(m).wait()
        scat_desc[m] = pltpu.async_copy(
            rows.at[m % NB], acc_sh.at[si4.at[m % NI]], ssem[m % NB],
            add=True)
    wait_scat(nch - 1)

    plsc.subcore_barrier()
    for off, sz in _WO:
        r = r0 + off
        pltpu.sync_copy(acc_sh.at[pl.ds(r, sz)], rows.at[0, pl.ds(0, sz)])
        pltpu.sync_copy(rows.at[0, pl.ds(0, sz)], out_at(r, sz))


def _segsum_scratch():
    return [
        pltpu.VMEM((NI, C_E), jnp.int32),
        pltpu.VMEM((NI, C_E), jnp.int32),
        pltpu.VMEM((NB, C_E, D), jnp.float32),
        pltpu.VMEM_SHARED((NP, D), jnp.float32),
    ] + [pltpu.SemaphoreType.DMA] * (2 * NI + 2 * NB)


def _make_segsum(n, e, d):
    assert NSTG * SE * NC * NS == e
    out_type = [jax.ShapeDtypeStruct((NC, NP, d), jnp.float32)]

    def body(t_hbm, g_hbm, s_hbm, out_hbm, gi_v, si2, rows, acc_sh, *sems):
        c = lax.axis_index("c")
        s = lax.axis_index("s")
        base_e = (c * NS + s) * EPT
        _segsum_phase(t_hbm, g_hbm, s_hbm,
                      lambda r, sz: out_hbm.at[c, pl.ds(r, sz)],
                      gi_v, si2, rows, acc_sh,
                      sems[0:2 * NI], sems[2 * NI:2 * NI + NB],
                      sems[2 * NI + NB:], base_e, EPT // C_E, s)

    return pl.kernel(body, out_type=out_type, mesh=_mesh(),
                     scratch_types=_segsum_scratch())


def _make_segsum2(n, e, d):
    ept2 = e // NS
    assert ept2 % C_E == 0
    out_type = [jax.ShapeDtypeStruct((NP, d), jnp.float32),
                jax.ShapeDtypeStruct((NP, d), jnp.float32)]

    def body(t1_hbm, t2_hbm, g_hbm, s_hbm, o1_hbm, o2_hbm,
             gi_v, si2, rows, acc_sh, *sems):
        c = lax.axis_index("c")
        s = lax.axis_index("s")
        base_e = s * ept2

        @pl.when(c == 0)
        def _():
            _segsum_phase(t1_hbm, g_hbm, s_hbm,
                          lambda r, sz: o1_hbm.at[pl.ds(r, sz)],
                          gi_v, si2, rows, acc_sh,
                          sems[0:2 * NI], sems[2 * NI:2 * NI + NB],
                          sems[2 * NI + NB:], base_e, ept2 // C_E, s)

        @pl.when(c == 1)
        def _():
            _segsum_phase(t2_hbm, g_hbm, s_hbm,
                          lambda r, sz: o2_hbm.at[pl.ds(r, sz)],
                          gi_v, si2, rows, acc_sh,
                          sems[0:2 * NI], sems[2 * NI:2 * NI + NB],
                          sems[2 * NI + NB:], base_e, ept2 // C_E, s)

    return pl.kernel(body, out_type=out_type, mesh=_mesh(),
                     scratch_types=_segsum_scratch())


def _make_counts(n, e):
    out_type = [jax.ShapeDtypeStruct((2, NC * NS, NPC), jnp.float32)]
    scratch = [
        pltpu.VMEM((SE,), jnp.int32),
        pltpu.VMEM((SE,), jnp.int32),
        pltpu.VMEM((NPC,), jnp.float32),
        pltpu.VMEM((NPC,), jnp.float32),
    ]

    def body(g_hbm, s_hbm, out_hbm, gi_v, si_v, csrc, cdst):
        c = lax.axis_index("c")
        s = lax.axis_index("s")
        z16 = jnp.zeros((L,), jnp.float32)
        one16 = jnp.ones((L,), jnp.float32)

        def zrow(i, _):
            csrc[pl.ds(i * L, L)] = z16
            cdst[pl.ds(i * L, L)] = z16
            return 0

        lax.fori_loop(0, NPC // L, zrow, 0)

        wid = c * NS + s
        base = wid * EPT
        for stg in range(NSTG):
            off = pl.multiple_of(base + stg * SE, 8)
            pltpu.sync_copy(g_hbm.at[pl.ds(off, SE)], gi_v)
            pltpu.sync_copy(s_hbm.at[pl.ds(off, SE)], si_v)

            def vec_body(j, _):
                gi = gi_v[pl.ds(j * L, L)]
                si = si_v[pl.ds(j * L, L)]
                plsc.addupdate_scatter(csrc, [gi], one16)
                plsc.addupdate_scatter(cdst, [si], one16)
                return 0

            lax.fori_loop(0, SE // L, vec_body, 0)
        pltpu.sync_copy(csrc, out_hbm.at[0, wid])
        pltpu.sync_copy(cdst, out_hbm.at[1, wid])

    return pl.kernel(body, out_type=out_type, mesh=_mesh(),
                     scratch_types=scratch,
                     compiler_params=pltpu.CompilerParams(
                         needs_layout_passes=False))


def _col_reduce(c_ref, blk):
    m = c_ref[...].reshape(NC * NS, blk)
    ones = jnp.ones((NC * NS, 1), jnp.float32)
    return lax.dot_general(m, ones, (((0,), (0,)), ((), ())),
                           preferred_element_type=jnp.float32)


def _make_tc1(n, d, blk):
    grid = ((n + blk - 1) // blk,)

    def tc1(a_ref, c_ref, x_ref, wl_ref, bl_ref, wr_ref, h_ref, h2_ref):
        asum = a_ref[0] + a_ref[1]
        cnt = _col_reduce(c_ref, blk)
        agg = asum / jnp.maximum(cnt, 1.0)
        dn = (((1,), (1,)), ((), ()))
        h = lax.dot_general(agg, wl_ref[...], dn,
                            preferred_element_type=jnp.float32)
        h = h + bl_ref[...]
        h = h + lax.dot_general(x_ref[...], wr_ref[...], dn,
                                preferred_element_type=jnp.float32)
        h = jnp.maximum(h, 0.0)
        h_ref[...] = h
        h2_ref[...] = h * h

    return pl.pallas_call(
        tc1,
        grid=grid,
        in_specs=[
            pl.BlockSpec((NC, blk, d), lambda i: (0, i, 0)),
            pl.BlockSpec((1, NC * NS, blk), lambda i: (1, 0, i)),
            pl.BlockSpec((blk, d), lambda i: (i, 0)),
            pl.BlockSpec((d, d), lambda i: (0, 0)),
            pl.BlockSpec((1, d), lambda i: (0, 0)),
            pl.BlockSpec((d, d), lambda i: (0, 0)),
        ],
        out_specs=[
            pl.BlockSpec((blk, d), lambda i: (i, 0)),
            pl.BlockSpec((blk, d), lambda i: (i, 0)),
        ],
        out_shape=[
            jax.ShapeDtypeStruct((n, d), jnp.float32),
            jax.ShapeDtypeStruct((n, d), jnp.float32),
        ],
    )


def _make_tc2(n, d, blk):
    grid = ((n + blk - 1) // blk,)

    def tc2(s1_ref, s2_ref, gc_ref, h_ref, h2_ref, o_ref):
        s1 = s1_ref[...]
        s2 = s2_ref[...]
        g = _col_reduce(gc_ref, blk)
        h = h_ref[...]
        gg = (g * h2_ref[...] - 2.0 * h * s1 + s2) / jnp.maximum(g, 1.0)
        o_ref[...] = jnp.tanh(gg)

    return pl.pallas_call(
        tc2,
        grid=grid,
        in_specs=[
            pl.BlockSpec((blk, d), lambda i: (i, 0)),
            pl.BlockSpec((blk, d), lambda i: (i, 0)),
            pl.BlockSpec((1, NC * NS, blk), lambda i: (0, 0, i)),
            pl.BlockSpec((blk, d), lambda i: (i, 0)),
            pl.BlockSpec((blk, d), lambda i: (i, 0)),
        ],
        out_specs=pl.BlockSpec((blk, d), lambda i: (i, 0)),
        out_shape=jax.ShapeDtypeStruct((n, d), jnp.float32),
    )


_make_segsum_c = functools.cache(_make_segsum)
_make_segsum2_c = functools.cache(_make_segsum2)
_make_counts_c = functools.cache(_make_counts)
_make_tc1_c = functools.cache(_make_tc1)
_make_tc2_c = functools.cache(_make_tc2)


def kernel(X, edge_index, W_l, b_l, W_r):
    src = edge_index[0]
    dst = edge_index[1]
    CNT, = _make_counts_c(N, E)(src, dst)
    P, = _make_segsum_c(N, E, D)(X, src, dst)
    H, H2 = _make_tc1_c(N, D, 1024)(P, CNT, X, W_l, b_l.reshape(1, D), W_r)
    S1, S2 = _make_segsum2_c(N, E, D)(H, H2, dst, src)
    return _make_tc2_c(N, D, 1024)(S1, S2, CNT, H, H2)
